# baseline (speedup 1.0000x reference)
import functools

import jax
import jax.numpy as jnp
from jax import lax
from jax.experimental import pallas as pl
from jax.experimental.pallas import tpu as pltpu
from jax.experimental.pallas import tpu_sc as plsc

N = 10000
D = 128
E = 320000
NC = 2
NS = 16
NW = NC * NS
C = 128
K = -(-E // (NW * C))
EPAD = NW * C * K
RSUB = 632
NROWS = NS * RSUB


def _make_sc_agg():
    mesh = plsc.VectorSubcoreMesh(core_axis_name="c", subcore_axis_name="s")

    @functools.partial(
        pl.kernel,
        mesh=mesh,
        out_type=jax.ShapeDtypeStruct((NC, NROWS, D), jnp.float32),
        scratch_types=[
            pltpu.VMEM((K, C), jnp.int32),
            pltpu.VMEM((K, C), jnp.int32),
            pltpu.VMEM((C, D), jnp.float32),
            pltpu.VMEM_SHARED((NROWS, D), jnp.float32),
            pltpu.SemaphoreType.DMA,
        ],
    )
    def agg(x_hbm, src_hbm, dst_hbm, zero_hbm, out_hbm,
            src_v, dst_v, rows_v, acc, sem):
        cid = lax.axis_index("c")
        sid = lax.axis_index("s")
        wid = cid * NS + sid
        pltpu.sync_copy(zero_hbm, acc.at[pl.ds(sid * RSUB, RSUB)])
        pltpu.sync_copy(src_hbm.at[wid], src_v)
        pltpu.sync_copy(dst_hbm.at[wid], dst_v)
        plsc.subcore_barrier()

        @pl.loop(0, K)
        def _(j):
            pltpu.async_copy(x_hbm.at[src_v.at[j]], rows_v, sem).wait()
            pltpu.sync_copy(rows_v, acc.at[dst_v.at[j]], add=True)

        plsc.subcore_barrier()
        pltpu.sync_copy(acc.at[pl.ds(sid * RSUB, RSUB)],
                        out_hbm.at[cid, pl.ds(sid * RSUB, RSUB)])

    return agg


_SC_AGG = _make_sc_agg()

_BLK = 2000


def _mlp_body(relu_out, x_ref, p0_ref, p1_ref, wa_ref, ba_ref, wb_ref, bb_ref,
              o_ref):
    h = x_ref[...] + p0_ref[0] + p1_ref[0]
    h = jnp.dot(h, wa_ref[...], preferred_element_type=jnp.float32)
    h = jnp.maximum(h + ba_ref[...], 0.0)
    h = jnp.dot(h, wb_ref[...], preferred_element_type=jnp.float32)
    h = h + bb_ref[...]
    if relu_out:
        h = jnp.maximum(h, 0.0)
    o_ref[...] = h


def _mlp(x, p, wa, ba, wb, bb, relu_out):
    row_spec = pl.BlockSpec((_BLK, D), lambda i: (i, 0))
    p0_spec = pl.BlockSpec((1, _BLK, D), lambda i: (0, i, 0))
    p1_spec = pl.BlockSpec((1, _BLK, D), lambda i: (1, i, 0))
    full_spec = pl.BlockSpec((D, D), lambda i: (0, 0))
    vec_spec = pl.BlockSpec((1, D), lambda i: (0, 0))
    return pl.pallas_call(
        functools.partial(_mlp_body, relu_out),
        grid=(N // _BLK,),
        in_specs=[row_spec, p0_spec, p1_spec,
                  full_spec, vec_spec, full_spec, vec_spec],
        out_specs=row_spec,
        out_shape=jax.ShapeDtypeStruct((N, D), jnp.float32),
    )(x, p, p, wa, ba.reshape(1, D), wb, bb.reshape(1, D))


def kernel(x, edge_index, W1a, b1a, W1b, b1b, W2a, b2a, W2b, b2b):
    src = edge_index[0]
    dst = edge_index[1]
    pad = EPAD - E
    src_p = jnp.concatenate([src, jnp.zeros((pad,), jnp.int32)]).reshape(NW, K, C)
    dst_p = jnp.concatenate([dst, jnp.full((pad,), N, jnp.int32)]).reshape(NW, K, C)
    zeros = jnp.zeros((RSUB, D), jnp.float32)

    p = _SC_AGG(x, src_p, dst_p, zeros)
    h = _mlp(x, p, W1a, b1a, W1b, b1b, True)
    p2 = _SC_AGG(h, src_p, dst_p, zeros)
    out = _mlp(h, p2, W2a, b2a, W2b, b2b, False)
    return out


# speedup vs baseline: 1.0302x; 1.0302x over previous
import functools

import jax
import jax.numpy as jnp
from jax import lax
from jax.experimental import pallas as pl
from jax.experimental.pallas import tpu as pltpu
from jax.experimental.pallas import tpu_sc as plsc

N = 10000
D = 128
E = 320000
NC = 2
NS = 16
NW = NC * NS
C = 128
K = -(-E // (NW * C))
EPAD = NW * C * K
RSUB = 632
NROWS = NS * RSUB


def _make_sc_agg():
    mesh = plsc.VectorSubcoreMesh(core_axis_name="c", subcore_axis_name="s")

    @functools.partial(
        pl.kernel,
        mesh=mesh,
        out_type=jax.ShapeDtypeStruct((NC, NROWS, D), jnp.float32),
        scratch_types=[
            pltpu.VMEM((K, C), jnp.int32),
            pltpu.VMEM((K, C), jnp.int32),
            pltpu.VMEM((C, D), jnp.float32),
            pltpu.VMEM_SHARED((NROWS, D), jnp.float32),
            pltpu.SemaphoreType.DMA,
        ],
    )
    def agg(x_hbm, src_hbm, dst_hbm, zero_hbm, out_hbm,
            src_v, dst_v, rows_v, acc, sem):
        cid = lax.axis_index("c")
        sid = lax.axis_index("s")
        wid = cid * NS + sid
        pltpu.sync_copy(zero_hbm.at[pl.ds(sid * RSUB, RSUB)],
                        acc.at[pl.ds(sid * RSUB, RSUB)])
        pltpu.sync_copy(src_hbm.at[wid], src_v)
        pltpu.sync_copy(dst_hbm.at[wid], dst_v)
        plsc.subcore_barrier()

        @pl.loop(0, K)
        def _(j):
            pltpu.async_copy(x_hbm.at[src_v.at[j]], rows_v, sem).wait()
            pltpu.sync_copy(rows_v, acc.at[dst_v.at[j]], add=True)

        plsc.subcore_barrier()
        pltpu.sync_copy(acc.at[pl.ds(sid * RSUB, RSUB)],
                        out_hbm.at[cid, pl.ds(sid * RSUB, RSUB)])

    return agg


_SC_AGG = _make_sc_agg()

_BLK = 2000


def _mlp_body(relu_out, x_ref, p0_ref, p1_ref, wa_ref, ba_ref, wb_ref, bb_ref,
              o_ref):
    h = x_ref[...] + p0_ref[0] + p1_ref[0]
    h = jnp.dot(h, wa_ref[...], preferred_element_type=jnp.float32)
    h = jnp.maximum(h + ba_ref[...], 0.0)
    h = jnp.dot(h, wb_ref[...], preferred_element_type=jnp.float32)
    h = h + bb_ref[...]
    if relu_out:
        h = jnp.maximum(h, 0.0)
    o_ref[...] = h


def _mlp(x, p, wa, ba, wb, bb, relu_out):
    row_spec = pl.BlockSpec((_BLK, D), lambda i: (i, 0))
    p0_spec = pl.BlockSpec((1, _BLK, D), lambda i: (0, i, 0))
    p1_spec = pl.BlockSpec((1, _BLK, D), lambda i: (1, i, 0))
    full_spec = pl.BlockSpec((D, D), lambda i: (0, 0))
    vec_spec = pl.BlockSpec((1, D), lambda i: (0, 0))
    return pl.pallas_call(
        functools.partial(_mlp_body, relu_out),
        grid=(N // _BLK,),
        in_specs=[row_spec, p0_spec, p1_spec,
                  full_spec, vec_spec, full_spec, vec_spec],
        out_specs=row_spec,
        out_shape=jax.ShapeDtypeStruct((N, D), jnp.float32),
    )(x, p, p, wa, ba.reshape(1, D), wb, bb.reshape(1, D))


def kernel(x, edge_index, W1a, b1a, W1b, b1b, W2a, b2a, W2b, b2b):
    src = edge_index[0]
    dst = edge_index[1]
    pad = EPAD - E
    src_p = jnp.concatenate([src, jnp.zeros((pad,), jnp.int32)]).reshape(NW, K, C)
    dst_p = jnp.concatenate([dst, jnp.full((pad,), N, jnp.int32)]).reshape(NW, K, C)
    zeros = jnp.zeros((NROWS, D), jnp.float32)

    p = _SC_AGG(x, src_p, dst_p, zeros)
    h = _mlp(x, p, W1a, b1a, W1b, b1b, True)
    p2 = _SC_AGG(h, src_p, dst_p, zeros)
    out = _mlp(h, p2, W2a, b2a, W2b, b2b, False)
    return out
